# Initial kernel scaffold; baseline (speedup 1.0000x reference)
import jax
import jax.numpy as jnp
from jax.experimental import pallas as pl


def kernel(row_x, token_x, t2r_edge_index, edge_attr_t2r, r2t_edge_index, edge_attr_r2t, W, b, gamma, beta):
    raise NotImplementedError("write your pallas kernel here")


# speedup vs baseline: 2.4693x; 2.4693x over previous
import functools

import jax
import jax.numpy as jnp
from jax import lax
from jax.experimental import pallas as pl
from jax.experimental.pallas import tpu as pltpu
from jax.experimental.pallas import tpu_sc as plsc


def _sc_accumulate(token2, edges, attr, z_a, z_b, z_c):
    n_rows = z_a.shape[0]
    e = edges.shape[0] // 2
    n_sub = 16
    chunk = 80
    ept = e // n_sub
    n_chunks = ept // chunk
    assert ept % chunk == 0 and n_rows % n_sub == 0
    rpt = n_rows // n_sub

    mesh = plsc.VectorSubcoreMesh(core_axis_name="c", subcore_axis_name="s")

    @functools.partial(
        pl.kernel,
        compiler_params=pltpu.CompilerParams(use_tc_tiling_on_sc=False),
        out_type=(
            jax.ShapeDtypeStruct((n_rows, 192), jnp.float32),
            jax.ShapeDtypeStruct((n_rows, 16), jnp.float32),
        ),
        mesh=mesh,
        scratch_types=[
            pltpu.VMEM_SHARED((n_rows, 64), jnp.float32),
            pltpu.VMEM_SHARED((n_rows, 32), jnp.float32),
            pltpu.VMEM_SHARED((n_rows, 16), jnp.float32),
            pltpu.VMEM((chunk,), jnp.int32),
            pltpu.VMEM((chunk,), jnp.int32),
            pltpu.VMEM((chunk,), jnp.int32),
            pltpu.VMEM((chunk, 64), jnp.float32),
            pltpu.VMEM((chunk, 32), jnp.float32),
            pltpu.VMEM((chunk, 16), jnp.float32),
            pltpu.SemaphoreType.DMA,
        ],
    )
    def sc_kernel(token2_h, edges_h, attr_h, za_h, zb_h, zc_h, acc_h, cnt_h,
                  acc_a, acc_b, acc_c, src_v, dst_v, idx_v, rows_v, attr_v,
                  ones_v, sem):
        c = lax.axis_index("c")
        s = lax.axis_index("s")
        r0 = s * rpt

        pltpu.sync_copy(za_h.at[pl.ds(r0, rpt)], acc_a.at[pl.ds(r0, rpt)])
        pltpu.sync_copy(zb_h.at[pl.ds(r0, rpt)], acc_b.at[pl.ds(r0, rpt)])

        @pl.when(c == 0)
        def _():
            pltpu.sync_copy(zc_h.at[pl.ds(r0, rpt)], acc_c.at[pl.ds(r0, rpt)])

        for j in range(chunk):
            ones_v[j] = jnp.full((16,), 1.0, jnp.float32)
        plsc.subcore_barrier()

        ebase = s * ept

        def body(i, carry):
            e0 = ebase + i * chunk
            pltpu.sync_copy(edges_h.at[pl.ds(e0, chunk)], src_v)
            pltpu.sync_copy(edges_h.at[pl.ds(e + e0, chunk)], dst_v)
            for j in range(chunk // 16):
                sl = pl.ds(j * 16, 16)
                idx_v[sl] = src_v[sl] * 2 + c
            pltpu.async_copy(token2_h.at[idx_v], rows_v, sem).wait()
            pltpu.sync_copy(attr_h.at[pl.ds(e0, chunk), pl.ds(c * 32, 32)],
                            attr_v)
            pltpu.sync_copy(rows_v, acc_a.at[dst_v], add=True)
            pltpu.sync_copy(attr_v, acc_b.at[dst_v], add=True)

            @pl.when(c == 0)
            def _():
                pltpu.sync_copy(ones_v, acc_c.at[dst_v], add=True)

            return carry

        lax.fori_loop(0, n_chunks, body, 0)
        plsc.subcore_barrier()

        rows = pl.ds(r0, rpt)
        pltpu.sync_copy(acc_a.at[rows], acc_h.at[rows, pl.ds(c * 64, 64)])
        pltpu.sync_copy(acc_b.at[rows], acc_h.at[rows, pl.ds(128 + c * 32, 32)])

        @pl.when(c == 0)
        def _():
            pltpu.sync_copy(acc_c.at[rows], cnt_h.at[rows])

    return sc_kernel(token2, edges, attr, z_a, z_b, z_c)


def _tc_body(acc_ref, cnt_ref, row_ref, w_ref, b_ref, g_ref, be_ref, out_ref):
    s = jnp.dot(acc_ref[...], w_ref[...], preferred_element_type=jnp.float32)
    cnt = cnt_ref[:, 0:1]
    msg = (s + cnt * b_ref[...]) / jnp.maximum(cnt, 1.0)
    x = row_ref[...] + msg
    mu = jnp.mean(x, axis=-1, keepdims=True)
    var = jnp.mean((x - mu) ** 2, axis=-1, keepdims=True)
    out_ref[...] = (x - mu) * lax.rsqrt(var + 1e-5) * g_ref[...] + be_ref[...]


def kernel(row_x, token_x, t2r_edge_index, edge_attr_t2r, r2t_edge_index,
           edge_attr_r2t, W, b, gamma, beta):
    n_rows, d = row_x.shape
    de = edge_attr_t2r.shape[1]
    n_pad = 10240
    token2 = token_x.reshape(-1, d // 2)
    z_a = jnp.zeros((n_pad, 64), jnp.float32)
    z_b = jnp.zeros((n_pad, 32), jnp.float32)
    z_c = jnp.zeros((n_pad, 16), jnp.float32)
    acc, cnt = _sc_accumulate(token2, t2r_edge_index.reshape(-1),
                              edge_attr_t2r, z_a, z_b, z_c)

    blk = 1000
    grid = n_rows // blk
    row_new = pl.pallas_call(
        _tc_body,
        grid=(grid,),
        in_specs=[
            pl.BlockSpec((blk, d + de), lambda i: (i, 0)),
            pl.BlockSpec((blk, 16), lambda i: (i, 0)),
            pl.BlockSpec((blk, d), lambda i: (i, 0)),
            pl.BlockSpec((d + de, d), lambda i: (0, 0)),
            pl.BlockSpec((1, d), lambda i: (0, 0)),
            pl.BlockSpec((1, d), lambda i: (0, 0)),
            pl.BlockSpec((1, d), lambda i: (0, 0)),
        ],
        out_specs=pl.BlockSpec((blk, d), lambda i: (i, 0)),
        out_shape=jax.ShapeDtypeStruct((n_rows, d), jnp.float32),
    )(acc, cnt, row_x, W, b.reshape(1, d), gamma.reshape(1, d),
      beta.reshape(1, d))
    return (row_new, token_x)


# speedup vs baseline: 4.2626x; 1.7262x over previous
import functools

import jax
import jax.numpy as jnp
from jax import lax
from jax.experimental import pallas as pl
from jax.experimental.pallas import tpu as pltpu
from jax.experimental.pallas import tpu_sc as plsc

_CHUNK = 128
_NSUB = 16


def _sc_accumulate(token2, idx2, dstp, attr, z_a, z_b, z_c):
    n_pad = z_a.shape[0]
    ep = dstp.shape[0]
    e_real = attr.shape[0]
    chunk = _CHUNK
    ept = ep // _NSUB
    n_chunks = ept // chunk
    assert ept % chunk == 0 and n_chunks % 2 == 0 and n_pad % _NSUB == 0
    rpt = n_pad // _NSUB

    mesh = plsc.VectorSubcoreMesh(core_axis_name="c", subcore_axis_name="s")

    @functools.partial(
        pl.kernel,
        compiler_params=pltpu.CompilerParams(use_tc_tiling_on_sc=False),
        out_type=(
            jax.ShapeDtypeStruct((n_pad, 192), jnp.float32),
            jax.ShapeDtypeStruct((n_pad, 16), jnp.float32),
        ),
        mesh=mesh,
        scratch_types=[
            pltpu.VMEM_SHARED((n_pad, 64), jnp.float32),
            pltpu.VMEM_SHARED((n_pad, 32), jnp.float32),
            pltpu.VMEM_SHARED((n_pad, 16), jnp.float32),
            pltpu.VMEM((chunk,), jnp.int32),
            pltpu.VMEM((chunk,), jnp.int32),
            pltpu.VMEM((chunk,), jnp.int32),
            pltpu.VMEM((chunk,), jnp.int32),
            pltpu.VMEM((chunk, 64), jnp.float32),
            pltpu.VMEM((chunk, 64), jnp.float32),
            pltpu.VMEM((chunk, 32), jnp.float32),
            pltpu.VMEM((chunk, 32), jnp.float32),
            pltpu.VMEM((chunk, 16), jnp.float32),
            pltpu.SemaphoreType.DMA,
            pltpu.SemaphoreType.DMA,
            pltpu.SemaphoreType.DMA,
            pltpu.SemaphoreType.DMA,
            pltpu.SemaphoreType.DMA,
            pltpu.SemaphoreType.DMA,
            pltpu.SemaphoreType.DMA,
            pltpu.SemaphoreType.DMA,
        ],
    )
    def sc_kernel(token2_h, idx2_h, dstp_h, attr_h, za_h, zb_h, zc_h,
                  acc_h, cnt_h, acc_a, acc_b, acc_c,
                  idx_v0, idx_v1, dst_v0, dst_v1, rows_v0, rows_v1,
                  attr_v0, attr_v1, ones_v,
                  sem_i0, sem_i1, sem_g0, sem_g1, sem_a0, sem_a1,
                  sem_s0, sem_s1):
        c = lax.axis_index("c")
        s = lax.axis_index("s")
        idx_v = (idx_v0, idx_v1)
        dst_v = (dst_v0, dst_v1)
        rows_v = (rows_v0, rows_v1)
        attr_v = (attr_v0, attr_v1)
        sem_i = (sem_i0, sem_i1)
        sem_g = (sem_g0, sem_g1)
        sem_a = (sem_a0, sem_a1)
        sem_s = (sem_s0, sem_s1)

        r0 = s * rpt
        rows = pl.ds(r0, rpt)
        pltpu.sync_copy(za_h.at[rows], acc_a.at[rows])
        pltpu.sync_copy(zb_h.at[rows], acc_b.at[rows])

        @pl.when(c == 0)
        def _():
            pltpu.sync_copy(zc_h.at[rows], acc_c.at[rows])

        for j in range(chunk):
            ones_v[j] = jnp.full((16,), 1.0, jnp.float32)
        plsc.subcore_barrier()

        ebase = s * ept
        idx_base = c * ep

        def issue_idx(i, b):
            e0 = ebase + i * chunk
            pltpu.async_copy(idx2_h.at[pl.ds(idx_base + e0, chunk)],
                             idx_v[b], sem_i[b])
            pltpu.async_copy(dstp_h.at[pl.ds(e0, chunk)], dst_v[b], sem_i[b])

        def wait_idx(b):
            pltpu.make_async_copy(idx2_h.at[pl.ds(0, chunk)], idx_v[b],
                                  sem_i[b]).wait()
            pltpu.make_async_copy(dstp_h.at[pl.ds(0, chunk)], dst_v[b],
                                  sem_i[b]).wait()

        def issue_ga(i, b):
            pltpu.async_copy(token2_h.at[idx_v[b]], rows_v[b], sem_g[b])
            a0 = jnp.minimum(ebase + i * chunk, e_real - chunk)
            pltpu.async_copy(attr_h.at[pl.ds(a0, chunk), pl.ds(c * 32, 32)],
                             attr_v[b], sem_a[b])

        def wait_ga(b):
            pltpu.make_async_copy(token2_h.at[idx_v[b]], rows_v[b],
                                  sem_g[b]).wait()
            pltpu.make_async_copy(attr_h.at[pl.ds(0, chunk), pl.ds(0, 32)],
                                  attr_v[b], sem_a[b]).wait()

        def issue_scatter(b):
            pltpu.async_copy(rows_v[b], acc_a.at[dst_v[b]], sem_s[b], add=True)
            pltpu.async_copy(attr_v[b], acc_b.at[dst_v[b]], sem_s[b], add=True)

            @pl.when(c == 0)
            def _():
                pltpu.async_copy(ones_v, acc_c.at[dst_v[b]], sem_s[b],
                                 add=True)

        def wait_scatter(b):
            pltpu.make_async_copy(rows_v[b], acc_a.at[dst_v[b]],
                                  sem_s[b]).wait()
            pltpu.make_async_copy(attr_v[b], acc_b.at[dst_v[b]],
                                  sem_s[b]).wait()

            @pl.when(c == 0)
            def _():
                pltpu.make_async_copy(ones_v, acc_c.at[dst_v[b]],
                                      sem_s[b]).wait()

        issue_idx(0, 0)
        wait_idx(0)
        issue_ga(0, 0)
        issue_idx(1, 1)

        def body(k, carry):
            for b in (0, 1):
                i = 2 * k + b
                b1 = 1 - b

                @pl.when(i + 1 < n_chunks)
                def _():
                    wait_idx(b1)
                    issue_ga(i + 1, b1)

                wait_ga(b)
                issue_scatter(b)
                wait_scatter(b)

                @pl.when(i + 2 < n_chunks)
                def _():
                    issue_idx(i + 2, b)

            return carry

        lax.fori_loop(0, n_chunks // 2, body, 0)
        plsc.subcore_barrier()

        pltpu.sync_copy(acc_a.at[rows], acc_h.at[rows, pl.ds(c * 64, 64)])
        pltpu.sync_copy(acc_b.at[rows], acc_h.at[rows, pl.ds(128 + c * 32, 32)])

        @pl.when(c == 0)
        def _():
            pltpu.sync_copy(acc_c.at[rows], cnt_h.at[rows])

    return sc_kernel(token2, idx2, dstp, attr, z_a, z_b, z_c)


def _tc_body(acc_ref, cnt_ref, row_ref, w_ref, b_ref, g_ref, be_ref, out_ref):
    s = jnp.dot(acc_ref[...], w_ref[...], preferred_element_type=jnp.float32)
    cnt = cnt_ref[:, 0:1]
    msg = (s + cnt * b_ref[...]) / jnp.maximum(cnt, 1.0)
    x = row_ref[...] + msg
    mu = jnp.mean(x, axis=-1, keepdims=True)
    var = jnp.mean((x - mu) ** 2, axis=-1, keepdims=True)
    out_ref[...] = (x - mu) * lax.rsqrt(var + 1e-5) * g_ref[...] + be_ref[...]


def kernel(row_x, token_x, t2r_edge_index, edge_attr_t2r, r2t_edge_index,
           edge_attr_r2t, W, b, gamma, beta):
    n_rows, d = row_x.shape
    de = edge_attr_t2r.shape[1]
    e = t2r_edge_index.shape[1]
    assert e % _CHUNK == 0
    n_pad = 10240

    cpt = -(-e // (_NSUB * _CHUNK))
    cpt += cpt % 2
    ep = cpt * _NSUB * _CHUNK
    pad = ep - e
    src = jnp.concatenate([t2r_edge_index[0], jnp.zeros((pad,), jnp.int32)])
    idx2 = jnp.concatenate([src * 2, src * 2 + 1])
    dstp = jnp.concatenate([t2r_edge_index[1],
                            jnp.full((pad,), n_rows, jnp.int32)])

    token2 = token_x.reshape(-1, d // 2)
    z_a = jnp.zeros((n_pad, 64), jnp.float32)
    z_b = jnp.zeros((n_pad, 32), jnp.float32)
    z_c = jnp.zeros((n_pad, 16), jnp.float32)
    acc, cnt = _sc_accumulate(token2, idx2, dstp, edge_attr_t2r,
                              z_a, z_b, z_c)

    blk = 1000
    grid = n_rows // blk
    row_new = pl.pallas_call(
        _tc_body,
        grid=(grid,),
        in_specs=[
            pl.BlockSpec((blk, d + de), lambda i: (i, 0)),
            pl.BlockSpec((blk, 16), lambda i: (i, 0)),
            pl.BlockSpec((blk, d), lambda i: (i, 0)),
            pl.BlockSpec((d + de, d), lambda i: (0, 0)),
            pl.BlockSpec((1, d), lambda i: (0, 0)),
            pl.BlockSpec((1, d), lambda i: (0, 0)),
            pl.BlockSpec((1, d), lambda i: (0, 0)),
        ],
        out_specs=pl.BlockSpec((blk, d), lambda i: (i, 0)),
        out_shape=jax.ShapeDtypeStruct((n_rows, d), jnp.float32),
    )(acc, cnt, row_x, W, b.reshape(1, d), gamma.reshape(1, d),
      beta.reshape(1, d))
    return (row_new, token_x)
